# Initial kernel scaffold; baseline (speedup 1.0000x reference)
import jax
import jax.numpy as jnp
from jax.experimental import pallas as pl


def kernel(x, edge_index, edge_weight, c, i, W1, b1, W2, b2, W3, b3, W4, b4):
    raise NotImplementedError("write your pallas kernel here")


# speedup vs baseline: 6.8024x; 6.8024x over previous
import functools
import jax
import jax.numpy as jnp
from jax import lax
from jax.experimental import pallas as pl
from jax.experimental.pallas import tpu as pltpu
from jax.experimental.pallas import tpu_sc as plsc

N_PAD = 50176
K_HOPS = 50


def _make_hop(n_pad, n_edges, fpad):
    info = plsc.get_sparse_core_info()
    nc, ns = info.num_cores, info.num_subcores
    nw = nc * ns
    epw = n_edges // nw
    C = 1600
    nchunks = epw // C
    rpt = n_pad // ns
    ZR = 392
    nz = rpt // ZR
    mesh = plsc.VectorSubcoreMesh(core_axis_name="c", subcore_axis_name="s")

    @functools.partial(
        pl.kernel, mesh=mesh,
        compiler_params=pltpu.CompilerParams(use_tc_tiling_on_sc=False),
        out_type=jax.ShapeDtypeStruct((nc * n_pad, fpad), jnp.float32),
        scratch_types=[
            pltpu.VMEM((C,), jnp.int32),
            pltpu.VMEM((C,), jnp.int32),
            pltpu.VMEM((C,), jnp.float32),
            pltpu.VMEM((C, fpad), jnp.float32),
            pltpu.VMEM((ZR, fpad), jnp.float32),
            pltpu.VMEM_SHARED((n_pad, fpad), jnp.float32),
            pltpu.SemaphoreType.DMA,
        ],
    )
    def hop(p_hbm, src_hbm, dst_hbm, ew_hbm, out_hbm,
            srcv, dstv, ewv, rows, zbuf, acc, sem):
        cid = lax.axis_index("c")
        sid = lax.axis_index("s")
        wid = sid * nc + cid
        zero16 = jnp.zeros((16,), jnp.float32)

        def zb(zi, carry):
            for f0 in range(0, fpad, 16):
                zbuf[zi, f0:f0 + 16] = zero16
            return carry
        lax.fori_loop(0, ZR, zb, 0)

        r0 = sid * rpt

        def zs(m, carry):
            pltpu.sync_copy(zbuf, acc.at[pl.ds(r0 + m * ZR, ZR)])
            return carry
        lax.fori_loop(0, nz, zs, 0)
        plsc.subcore_barrier()

        ebase = wid * epw

        def chunk(j, carry):
            b = ebase + j * C
            pltpu.sync_copy(src_hbm.at[pl.ds(b, C)], srcv)
            pltpu.sync_copy(dst_hbm.at[pl.ds(b, C)], dstv)
            pltpu.sync_copy(ew_hbm.at[pl.ds(b, C)], ewv)
            pltpu.async_copy(p_hbm.at[srcv], rows, sem).wait()

            def scale(g, c2):
                e0 = g * 16
                ew16 = ewv[pl.ds(e0, 16)]
                for u in range(16):
                    wgt = ew16[u]
                    for f0 in range(0, fpad, 16):
                        rows[e0 + u, f0:f0 + 16] = rows[e0 + u, f0:f0 + 16] * wgt
                return c2
            lax.fori_loop(0, C // 16, scale, 0)
            pltpu.sync_copy(rows, acc.at[dstv], add=True)
            return carry
        lax.fori_loop(0, nchunks, chunk, 0)
        plsc.subcore_barrier()
        pltpu.sync_copy(acc.at[pl.ds(r0, rpt)],
                        out_hbm.at[pl.ds(cid * n_pad + r0, rpt)])

    def run(p, src, dst, ew):
        out = hop(p, src, dst, ew)
        return out.reshape(nc, n_pad, fpad).sum(axis=0)
    return run


def _mm_relu(xc, wc, bc):
    np_, kd = xc.shape
    R = 448

    def body(xr, wr, br, outr):
        acc = jnp.dot(xr[...], wr[...], preferred_element_type=jnp.float32)
        outr[...] = jnp.maximum(acc + br[...], 0.0)

    return pl.pallas_call(
        body,
        grid=(np_ // R,),
        in_specs=[
            pl.BlockSpec((R, kd), lambda i: (i, 0)),
            pl.BlockSpec((kd, 128), lambda i: (0, 0)),
            pl.BlockSpec((1, 128), lambda i: (0, 0)),
        ],
        out_specs=pl.BlockSpec((R, 128), lambda i: (i, 0)),
        out_shape=jax.ShapeDtypeStruct((np_, 128), jnp.float32),
    )(xc, wc, bc)


def _layer(h, dis, hop_run, src, dst, ew, W, b, fpad):
    n_pad = h.shape[0]
    fin, g = W.shape[1], W.shape[2]
    hp = jnp.pad(h, ((0, 0), (0, fpad - fin)))

    def step(p, _):
        q = dis[:, None] * p
        t = hop_run(q, src, dst, ew)
        pn = dis[:, None] * t
        return pn, pn

    _, ps = lax.scan(step, hp, None, length=K_HOPS)
    xc = jnp.concatenate([hp[None], ps], axis=0)
    xc = xc.transpose(1, 0, 2).reshape(n_pad, (K_HOPS + 1) * fpad)
    kd = ((xc.shape[1] + 127) // 128) * 128
    xc = jnp.pad(xc, ((0, 0), (0, kd - xc.shape[1])))
    wp = jnp.pad(W, ((0, 0), (0, fpad - fin), (0, 0)))
    wp = wp.reshape((K_HOPS + 1) * fpad, g)
    wp = jnp.pad(wp, ((0, kd - wp.shape[0]), (0, 128 - g)))
    bp = jnp.pad(b, (0, 128 - g)).reshape(1, 128)
    out = _mm_relu(xc, wp, bp)
    return out[:, :g]


def kernel(x, edge_index, edge_weight, c, i, W1, b1, W2, b2, W3, b3, W4, b4):
    n = c.shape[0]
    e0 = edge_weight.shape[0]
    e = ((e0 + 51199) // 51200) * 51200
    src = jnp.pad(edge_index[0], (0, e - e0))
    dst = jnp.pad(edge_index[1], (0, e - e0))
    edge_weight = jnp.pad(edge_weight, (0, e - e0))

    hop16 = _make_hop(N_PAD, e, 16)

    def hop32(q, s_, d_, w_):
        return jnp.concatenate(
            [hop16(q[:, 0:16], s_, d_, w_), hop16(q[:, 16:32], s_, d_, w_)],
            axis=1)

    ones = jnp.ones((N_PAD, 16), jnp.float32)
    deg = hop16(ones, src, dst, edge_weight)[:, 0]
    dis = jnp.where(deg > 0, jax.lax.rsqrt(jnp.where(deg > 0, deg, 1.0)), 0.0)

    cv = jnp.zeros((N_PAD,), jnp.float32).at[i].set(1.0)
    cp = jnp.pad(c.reshape(-1), (0, N_PAD - n))
    h = jnp.stack([cv, cp], axis=1)

    h = _layer(h, dis, hop16, src, dst, edge_weight, W1, b1, 16)
    h = _layer(h, dis, hop16, src, dst, edge_weight, W2, b2, 16)
    h = _layer(h, dis, hop32, src, dst, edge_weight, W3, b3, 32)
    h = _layer(h, dis, hop16, src, dst, edge_weight, W4, b4, 16)
    return h[:n, 0]


# speedup vs baseline: 7.0930x; 1.0427x over previous
import functools
import jax
import jax.numpy as jnp
from jax import lax
from jax.experimental import pallas as pl
from jax.experimental.pallas import tpu as pltpu
from jax.experimental.pallas import tpu_sc as plsc

N_PAD = 50176
K_HOPS = 50


def _make_hop(n_pad, n_edges, fpad):
    info = plsc.get_sparse_core_info()
    nc, ns = info.num_cores, info.num_subcores
    nw = nc * ns
    epw = n_edges // nw
    C = 800
    nchunks = epw // C
    rpt = n_pad // ns
    ZR = 392
    nz = rpt // ZR
    mesh = plsc.VectorSubcoreMesh(core_axis_name="c", subcore_axis_name="s")

    @functools.partial(
        pl.kernel, mesh=mesh,
        compiler_params=pltpu.CompilerParams(use_tc_tiling_on_sc=False),
        out_type=jax.ShapeDtypeStruct((nc * n_pad, fpad), jnp.float32),
        scratch_types=[
            pltpu.VMEM((2, C), jnp.int32),
            pltpu.VMEM((2, C), jnp.int32),
            pltpu.VMEM((2, C), jnp.float32),
            pltpu.VMEM((2, C, fpad), jnp.float32),
            pltpu.VMEM((ZR, fpad), jnp.float32),
            pltpu.VMEM_SHARED((n_pad, fpad), jnp.float32),
            pltpu.SemaphoreType.DMA,
            pltpu.SemaphoreType.DMA,
        ],
    )
    def hop(p_hbm, src_hbm, dst_hbm, ew_hbm, out_hbm,
            srcv, dstv, ewv, rows, zbuf, acc, sem_a, sem_b):
        cid = lax.axis_index("c")
        sid = lax.axis_index("s")
        wid = sid * nc + cid
        zero16 = jnp.zeros((16,), jnp.float32)

        def zb(zi, carry):
            for f0 in range(0, fpad, 16):
                zbuf[zi, f0:f0 + 16] = zero16
            return carry
        lax.fori_loop(0, ZR, zb, 0)

        r0 = sid * rpt

        def zs(m, carry):
            pltpu.sync_copy(zbuf, acc.at[pl.ds(r0 + m * ZR, ZR)])
            return carry
        lax.fori_loop(0, nz, zs, 0)
        plsc.subcore_barrier()

        ebase = wid * epw
        sems = (sem_a, sem_b)

        def load_idx_and_gather(j, par):
            b = ebase + j * C
            pltpu.sync_copy(src_hbm.at[pl.ds(b, C)], srcv.at[par])
            pltpu.sync_copy(dst_hbm.at[pl.ds(b, C)], dstv.at[par])
            pltpu.sync_copy(ew_hbm.at[pl.ds(b, C)], ewv.at[par])
            pltpu.async_copy(p_hbm.at[srcv.at[par]], rows.at[par], sems[par])

        load_idx_and_gather(0, 0)

        def step(s, carry):
            for par in range(2):
                j = s * 2 + par
                pltpu.make_async_copy(
                    p_hbm.at[srcv.at[par]], rows.at[par], sems[par]).wait()

                @pl.when(j + 1 < nchunks)
                def _prefetch():
                    load_idx_and_gather(j + 1, 1 - par)

                def scale(g, c2):
                    e0 = g * 16
                    ew16 = ewv[par, pl.ds(e0, 16)]
                    for u in range(16):
                        wgt = ew16[u]
                        for f0 in range(0, fpad, 16):
                            rows[par, e0 + u, f0:f0 + 16] = (
                                rows[par, e0 + u, f0:f0 + 16] * wgt)
                    return c2
                lax.fori_loop(0, C // 16, scale, 0)
                pltpu.sync_copy(rows.at[par], acc.at[dstv.at[par]], add=True)
            return carry
        lax.fori_loop(0, nchunks // 2, step, 0)
        plsc.subcore_barrier()
        pltpu.sync_copy(acc.at[pl.ds(r0, rpt)],
                        out_hbm.at[pl.ds(cid * n_pad + r0, rpt)])

    def run(p, src, dst, ew):
        out = hop(p, src, dst, ew)
        return out.reshape(nc, n_pad, fpad).sum(axis=0)
    return run


def _mm_relu(xc, wc, bc):
    np_, kd = xc.shape
    R = 448

    def body(xr, wr, br, outr):
        acc = jnp.dot(xr[...], wr[...], preferred_element_type=jnp.float32)
        outr[...] = jnp.maximum(acc + br[...], 0.0)

    return pl.pallas_call(
        body,
        grid=(np_ // R,),
        in_specs=[
            pl.BlockSpec((R, kd), lambda i: (i, 0)),
            pl.BlockSpec((kd, 128), lambda i: (0, 0)),
            pl.BlockSpec((1, 128), lambda i: (0, 0)),
        ],
        out_specs=pl.BlockSpec((R, 128), lambda i: (i, 0)),
        out_shape=jax.ShapeDtypeStruct((np_, 128), jnp.float32),
    )(xc, wc, bc)


def _layer(h, dis, hop_run, src, dst, ew, W, b, fpad):
    n_pad = h.shape[0]
    fin, g = W.shape[1], W.shape[2]
    hp = jnp.pad(h, ((0, 0), (0, fpad - fin)))

    def step(p, _):
        q = dis[:, None] * p
        t = hop_run(q, src, dst, ew)
        pn = dis[:, None] * t
        return pn, pn

    _, ps = lax.scan(step, hp, None, length=K_HOPS)
    xc = jnp.concatenate([hp[None], ps], axis=0)
    xc = xc.transpose(1, 0, 2).reshape(n_pad, (K_HOPS + 1) * fpad)
    kd = ((xc.shape[1] + 127) // 128) * 128
    xc = jnp.pad(xc, ((0, 0), (0, kd - xc.shape[1])))
    wp = jnp.pad(W, ((0, 0), (0, fpad - fin), (0, 0)))
    wp = wp.reshape((K_HOPS + 1) * fpad, g)
    wp = jnp.pad(wp, ((0, kd - wp.shape[0]), (0, 128 - g)))
    bp = jnp.pad(b, (0, 128 - g)).reshape(1, 128)
    out = _mm_relu(xc, wp, bp)
    return out[:, :g]


def kernel(x, edge_index, edge_weight, c, i, W1, b1, W2, b2, W3, b3, W4, b4):
    n = c.shape[0]
    e0 = edge_weight.shape[0]
    e = ((e0 + 51199) // 51200) * 51200
    src = jnp.pad(edge_index[0], (0, e - e0))
    dst = jnp.pad(edge_index[1], (0, e - e0))
    edge_weight = jnp.pad(edge_weight, (0, e - e0))

    hop16 = _make_hop(N_PAD, e, 16)

    def hop32(q, s_, d_, w_):
        return jnp.concatenate(
            [hop16(q[:, 0:16], s_, d_, w_), hop16(q[:, 16:32], s_, d_, w_)],
            axis=1)

    ones = jnp.ones((N_PAD, 16), jnp.float32)
    deg = hop16(ones, src, dst, edge_weight)[:, 0]
    dis = jnp.where(deg > 0, jax.lax.rsqrt(jnp.where(deg > 0, deg, 1.0)), 0.0)

    cv = jnp.zeros((N_PAD,), jnp.float32).at[i].set(1.0)
    cp = jnp.pad(c.reshape(-1), (0, N_PAD - n))
    h = jnp.stack([cv, cp], axis=1)

    h = _layer(h, dis, hop16, src, dst, edge_weight, W1, b1, 16)
    h = _layer(h, dis, hop16, src, dst, edge_weight, W2, b2, 16)
    h = _layer(h, dis, hop32, src, dst, edge_weight, W3, b3, 32)
    h = _layer(h, dis, hop16, src, dst, edge_weight, W4, b4, 16)
    return h[:n, 0]
